# Initial kernel scaffold; baseline (speedup 1.0000x reference)
import jax
import jax.numpy as jnp
from jax.experimental import pallas as pl


def kernel(x, edge_index, batch, W1, a_src1, a_dst1, b1, W2, a_src2, a_dst2, b2, W3, a_src3, a_dst3, b3, lin1_W, lin1_b, lin2_W, lin2_b):
    raise NotImplementedError("write your pallas kernel here")


# speedup vs baseline: 13.9664x; 13.9664x over previous
import functools
import jax
import jax.numpy as jnp
from jax import lax
from jax.experimental import pallas as pl
from jax.experimental.pallas import tpu as pltpu
from jax.experimental.pallas import tpu_sc as plsc

N = 10000
E_RAW = 320000
E_REAL = E_RAW + N
D = 128
G = 64
NC_OUT = 32

NCORE = 2
NSUB = 16
NW = NCORE * NSUB
CH = 128
E_PAD = ((E_REAL + NW * CH - 1) // (NW * CH)) * (NW * CH)
C1 = E_PAD // (NSUB * CH)
C2 = E_PAD // (NW * CH)
NPAD = 10240


def _sc_edge_kernel(h, asrc, adst, src, dst):
    mesh = plsc.VectorSubcoreMesh(core_axis_name="c", subcore_axis_name="s")

    @functools.partial(
        pl.kernel,
        mesh=mesh,
        out_type=[jax.ShapeDtypeStruct((2 * N, D), jnp.float32),
                  jax.ShapeDtypeStruct((2 * NPAD,), jnp.float32)],
        scratch_types=[
            pltpu.VMEM_SHARED((NPAD,), jnp.float32),
            pltpu.VMEM_SHARED((N, D), jnp.float32),
            pltpu.VMEM((16, D), jnp.float32),
            pltpu.VMEM((CH,), jnp.int32),
            pltpu.VMEM((CH,), jnp.int32),
            pltpu.VMEM((CH,), jnp.float32),
            pltpu.VMEM((CH, D), jnp.float32),
            pltpu.VMEM((CH,), jnp.float32),
            pltpu.VMEM((CH,), jnp.float32),
            pltpu.VMEM((CH,), jnp.float32),
            pltpu.VMEM((CH,), jnp.int32),
            pltpu.SemaphoreType.DMA,
        ],
    )
    def k(h_h, asrc_h, adst_h, src_h, dst_h, out_h, dn_h,
          denom_s, out_s, zbuf, src_v, dst_v, p_v, rows_v,
          a1_v, a2_v, dn_v, dst2_v, sem):
        cid = lax.axis_index("c")
        sid = lax.axis_index("s")
        wid = sid * NCORE + cid
        iota16 = lax.broadcasted_iota(jnp.int32, (16,), 0)
        z16 = jnp.zeros((16,), jnp.float32)

        for r in range(16):
            for j in range(8):
                zbuf[r, pl.ds(j * 16, 16)] = z16
        for kk in range(5):
            pltpu.sync_copy(zbuf.at[0],
                            denom_s.at[pl.ds(sid * 640 + kk * 128, 128)])
        @pl.when(sid < 15)
        def _():
            for kk in range(40):
                pltpu.sync_copy(
                    zbuf, out_s.at[pl.ds(sid * 640 + kk * 16, 16)])

        @pl.when(sid == 15)
        def _():
            for kk in range(25):
                pltpu.sync_copy(
                    zbuf, out_s.at[pl.ds(9600 + kk * 16, 16)])

        plsc.subcore_barrier()
        ones16 = jnp.ones((16,), jnp.float32)

        def p1_body(c, carry):
            off = sid * (C1 * CH) + c * CH
            pltpu.sync_copy(src_h.at[pl.ds(off, CH)], src_v)
            pltpu.sync_copy(dst_h.at[pl.ds(off, CH)], dst_v)
            c1 = pltpu.async_copy(asrc_h.at[src_v], a1_v, sem)
            c2 = pltpu.async_copy(adst_h.at[dst_v], a2_v, sem)
            c1.wait()
            c2.wait()
            for j in range(8):
                sl = pl.ds(j * 16, 16)
                e = a1_v[sl] + a2_v[sl]
                e = jnp.maximum(e, 0.2 * e)
                p = jnp.exp(e)
                g = off + j * 16 + iota16
                p_v[sl] = jnp.where(g < E_REAL, p, 0.0)
            pltpu.sync_copy(p_v, denom_s.at[dst_v], add=True)
            return carry

        lax.fori_loop(0, C1, p1_body, 0)
        plsc.subcore_barrier()
        pltpu.sync_copy(denom_s.at[pl.ds(sid * 640, 640)],
                        dn_h.at[pl.ds(cid * NPAD + sid * 640, 640)])
        plsc.subcore_barrier()

        def p2_body(c, carry):
            off = wid * (C2 * CH) + c * CH
            pltpu.sync_copy(src_h.at[pl.ds(off, CH)], src_v)
            pltpu.sync_copy(dst_h.at[pl.ds(off, CH)], dst_v)
            for j in range(8):
                sl = pl.ds(j * 16, 16)
                dst2_v[sl] = dst_v[sl] + cid * NPAD
            c0 = pltpu.async_copy(h_h.at[src_v], rows_v, sem)
            c1 = pltpu.async_copy(asrc_h.at[src_v], a1_v, sem)
            c2 = pltpu.async_copy(adst_h.at[dst_v], a2_v, sem)
            c3 = pltpu.async_copy(dn_h.at[dst2_v], dn_v, sem)
            c0.wait()
            c1.wait()
            c2.wait()
            c3.wait()
            for j in range(8):
                sl = pl.ds(j * 16, 16)
                e = a1_v[sl] + a2_v[sl]
                e = jnp.maximum(e, 0.2 * e)
                p = jnp.exp(e)
                g = off + j * 16 + iota16
                al = p / (dn_v[sl] + 1e-16)
                p_v[sl] = jnp.where(g < E_REAL, al, 0.0)

            def grp_body(g2, carry2):
                base_r = g2 * 16
                av_vec = p_v[pl.ds(base_r, 16)]
                for ln in range(16):
                    av = jnp.zeros((16,), jnp.float32) + av_vec[ln]
                    i = base_r + ln
                    for jj in range(8):
                        sl = pl.ds(jj * 16, 16)
                        rows_v[i, sl] = rows_v[i, sl] * av
                return carry2

            lax.fori_loop(0, CH // 16, grp_body, 0)
            pltpu.sync_copy(rows_v, out_s.at[dst_v], add=True)
            return carry

        lax.fori_loop(0, C2, p2_body, 0)
        plsc.subcore_barrier()

        @pl.when(sid < 15)
        def _():
            pltpu.sync_copy(
                out_s.at[pl.ds(sid * 640, 640)],
                out_h.at[pl.ds(cid * N + sid * 640, 640)])

        @pl.when(sid == 15)
        def _():
            pltpu.sync_copy(out_s.at[pl.ds(9600, 400)],
                            out_h.at[pl.ds(cid * N + 9600, 400)])

    out, _ = k(h, asrc, adst, src, dst)
    return out


def _mm1_body(x_ref, w_ref, a_ref, h_ref, s_ref):
    h = jnp.dot(x_ref[...], w_ref[...], preferred_element_type=jnp.float32)
    h_ref[...] = h
    s_ref[...] = jnp.dot(h, a_ref[...], preferred_element_type=jnp.float32)


def _mml_body(p0_ref, p1_ref, b_ref, w_ref, a_ref, h_ref, s_ref):
    x = jnp.maximum(p0_ref[...] + p1_ref[...] + b_ref[...], 0.0)
    h = jnp.dot(x, w_ref[...], preferred_element_type=jnp.float32)
    h_ref[...] = h
    s_ref[...] = jnp.dot(h, a_ref[...], preferred_element_type=jnp.float32)


_BLK = 1000
_NBLK = N // _BLK


def _tc_pre1(x, w, a2):
    return pl.pallas_call(
        _mm1_body,
        grid=(_NBLK,),
        in_specs=[
            pl.BlockSpec((_BLK, D), lambda i: (i, 0)),
            pl.BlockSpec((D, D), lambda i: (0, 0)),
            pl.BlockSpec((D, D), lambda i: (0, 0)),
        ],
        out_specs=[
            pl.BlockSpec((_BLK, D), lambda i: (i, 0)),
            pl.BlockSpec((_BLK, D), lambda i: (i, 0)),
        ],
        out_shape=[
            jax.ShapeDtypeStruct((N, D), jnp.float32),
            jax.ShapeDtypeStruct((N, D), jnp.float32),
        ],
    )(x, w, a2)


def _tc_prel(p, b, w, a2):
    return pl.pallas_call(
        _mml_body,
        grid=(_NBLK,),
        in_specs=[
            pl.BlockSpec((_BLK, D), lambda i: (i, 0)),
            pl.BlockSpec((_BLK, D), lambda i: (i + _NBLK, 0)),
            pl.BlockSpec((1, D), lambda i: (0, 0)),
            pl.BlockSpec((D, D), lambda i: (0, 0)),
            pl.BlockSpec((D, D), lambda i: (0, 0)),
        ],
        out_specs=[
            pl.BlockSpec((_BLK, D), lambda i: (i, 0)),
            pl.BlockSpec((_BLK, D), lambda i: (i, 0)),
        ],
        out_shape=[
            jax.ShapeDtypeStruct((N, D), jnp.float32),
            jax.ShapeDtypeStruct((N, D), jnp.float32),
        ],
    )(p, p, b, w, a2)


def _final_body(p0_ref, p1_ref, b_ref, oh_ref, l1w_ref, l1b_ref,
                l2w_ref, l2b_ref, out_ref, pool_acc, cnt_acc):
    i = pl.program_id(0)

    @pl.when(i == 0)
    def _():
        pool_acc[...] = jnp.zeros_like(pool_acc)
        cnt_acc[...] = jnp.zeros_like(cnt_acc)

    x = jnp.maximum(p0_ref[...] + p1_ref[...] + b_ref[...], 0.0)
    oh = oh_ref[0]
    pool_acc[...] += jnp.dot(oh, x, preferred_element_type=jnp.float32)
    cnt_acc[...] += jnp.dot(oh, jnp.ones((_BLK, D), jnp.float32),
                            preferred_element_type=jnp.float32)

    @pl.when(i == _NBLK - 1)
    def _():
        pooled = pool_acc[...] / jnp.maximum(cnt_acc[...], 1.0)
        h = jnp.maximum(
            jnp.dot(pooled, l1w_ref[...], preferred_element_type=jnp.float32)
            + l1b_ref[...], 0.0)
        o = jnp.dot(h, l2w_ref[...],
                    preferred_element_type=jnp.float32) + l2b_ref[...]
        m = jnp.max(o, axis=0, keepdims=True)
        lse = jnp.log(jnp.sum(jnp.exp(o - m), axis=0, keepdims=True)) + m
        out_ref[...] = o - lse


def _tc_final(p, b3, oh, l1w, l1b, l2w, l2b):
    return pl.pallas_call(
        _final_body,
        grid=(_NBLK,),
        in_specs=[
            pl.BlockSpec((_BLK, D), lambda i: (i, 0)),
            pl.BlockSpec((_BLK, D), lambda i: (i + _NBLK, 0)),
            pl.BlockSpec((1, D), lambda i: (0, 0)),
            pl.BlockSpec((1, G, _BLK), lambda i: (i, 0, 0)),
            pl.BlockSpec((D, D), lambda i: (0, 0)),
            pl.BlockSpec((1, D), lambda i: (0, 0)),
            pl.BlockSpec((D, D), lambda i: (0, 0)),
            pl.BlockSpec((1, D), lambda i: (0, 0)),
        ],
        out_specs=pl.BlockSpec((G, D), lambda i: (0, 0)),
        out_shape=jax.ShapeDtypeStruct((G, D), jnp.float32),
        scratch_shapes=[
            pltpu.VMEM((G, D), jnp.float32),
            pltpu.VMEM((G, D), jnp.float32),
        ],
    )(p, p, b3, oh, l1w, l1b, l2w, l2b)


def _pack_a(a_s, a_d):
    a2 = jnp.zeros((D, D), jnp.float32)
    return a2.at[:, 0].set(a_s).at[:, 1].set(a_d)


def kernel(x, edge_index, batch, W1, a_src1, a_dst1, b1, W2, a_src2, a_dst2,
           b2, W3, a_src3, a_dst3, b3, lin1_W, lin1_b, lin2_W, lin2_b):
    loop = jnp.arange(N, dtype=edge_index.dtype)
    pad = jnp.zeros((E_PAD - E_REAL,), edge_index.dtype)
    src = jnp.concatenate([edge_index[0], loop, pad])
    dst = jnp.concatenate([edge_index[1], loop, pad])

    oh = (batch[None, :] == jnp.arange(G, dtype=batch.dtype)[:, None])
    oh = oh.astype(jnp.float32).reshape(G, _NBLK, _BLK)
    oh = oh.transpose(1, 0, 2)

    h1, s1 = _tc_pre1(x, W1, _pack_a(a_src1, a_dst1))
    npad = jnp.zeros((NPAD - N,), jnp.float32)
    p1 = _sc_edge_kernel(h1, jnp.concatenate([s1[:, 0], npad]),
                         jnp.concatenate([s1[:, 1], npad]), src, dst)

    h2, s2 = _tc_prel(p1, b1.reshape(1, D), W2, _pack_a(a_src2, a_dst2))
    p2 = _sc_edge_kernel(h2, jnp.concatenate([s2[:, 0], npad]),
                         jnp.concatenate([s2[:, 1], npad]), src, dst)

    h3, s3 = _tc_prel(p2, b2.reshape(1, D), W3, _pack_a(a_src3, a_dst3))
    p3 = _sc_edge_kernel(h3, jnp.concatenate([s3[:, 0], npad]),
                         jnp.concatenate([s3[:, 1], npad]), src, dst)

    l2w = jnp.zeros((D, D), jnp.float32).at[:, :NC_OUT].set(lin2_W)
    l2b = jnp.zeros((1, D), jnp.float32).at[0, :NC_OUT].set(lin2_b)
    out = _tc_final(p3, b3.reshape(1, D), oh, lin1_W,
                    lin1_b.reshape(1, D), l2w, l2b)
    return out[:, :NC_OUT]
